# baseline (speedup 1.0000x reference)
import functools

import jax
import jax.numpy as jnp
from jax import lax
from jax.experimental import pallas as pl
from jax.experimental.pallas import tpu as pltpu
from jax.experimental.pallas import tpu_sc as plsc

VOCAB = 1000000
EMBED_DIM = 64
BATCH = 16384
NUM_CLASSES = 128

NW = 32
CHUNK = 128
NCHUNK = BATCH // NW // CHUNK
BM = 2048
NB = BATCH // BM


def _sc_gather(table, idx3):
    mesh = plsc.VectorSubcoreMesh(core_axis_name="c", subcore_axis_name="s")

    @functools.partial(
        pl.kernel,
        out_type=jax.ShapeDtypeStruct((NW, NCHUNK, CHUNK, EMBED_DIM), jnp.float32),
        mesh=mesh,
        scratch_types=[
            pltpu.VMEM((NCHUNK, CHUNK), jnp.int32),
            pltpu.VMEM((NCHUNK, CHUNK, EMBED_DIM), jnp.float32),
            pltpu.SemaphoreType.DMA,
        ],
        compiler_params=pltpu.CompilerParams(use_tc_tiling_on_sc=False),
    )
    def k(table_hbm, idx_hbm, out_hbm, idx_v, rows_v, sem):
        wid = lax.axis_index("s") * 2 + lax.axis_index("c")
        pltpu.sync_copy(idx_hbm.at[wid], idx_v)
        copies = [
            pltpu.make_async_copy(table_hbm.at[idx_v.at[j]], rows_v.at[j], sem)
            for j in range(NCHUNK)
        ]
        for c in copies:
            c.start()
        for c in copies:
            c.wait()
        pltpu.sync_copy(rows_v, out_hbm.at[wid])

    return k(table, idx3)


def _tc_body(x_ref, w_ref, b_ref, lab_ref, out_ref):
    i = pl.program_id(0)
    x = x_ref[...]
    w = w_ref[...]
    bias = b_ref[...]
    lab = lab_ref[0, 0, :]
    scores = jnp.dot(x, w, preferred_element_type=jnp.float32) + bias
    m = jnp.max(scores, axis=-1, keepdims=True)
    lse = jnp.log(jnp.sum(jnp.exp(scores - m), axis=-1, keepdims=True)) + m
    cls = lax.broadcasted_iota(jnp.int32, scores.shape, 1)
    picked = jnp.sum(
        jnp.where(cls == lab[:, None], scores, 0.0), axis=-1, keepdims=True
    )
    part = jnp.sum(lse - picked, axis=0, keepdims=True) * (1.0 / BATCH)

    @pl.when(i == 0)
    def _():
        out_ref[...] = part

    @pl.when(i > 0)
    def _():
        out_ref[...] = out_ref[...] + part


def _tc_loss(x, w, bias, labels3):
    return pl.pallas_call(
        _tc_body,
        grid=(NB,),
        in_specs=[
            pl.BlockSpec((BM, EMBED_DIM), lambda i: (i, 0)),
            pl.BlockSpec((EMBED_DIM, NUM_CLASSES), lambda i: (0, 0)),
            pl.BlockSpec((1, NUM_CLASSES), lambda i: (0, 0)),
            pl.BlockSpec((1, 1, BM), lambda i: (i, 0, 0)),
        ],
        out_specs=pl.BlockSpec((1, 1), lambda i: (0, 0)),
        out_shape=jax.ShapeDtypeStruct((1, 1), jnp.float32),
    )(x, w, bias, labels3)


def kernel(sentence_features, labels, emb, W, b):
    idx3 = sentence_features.astype(jnp.int32).reshape(NW, NCHUNK, CHUNK)
    gathered = _sc_gather(emb, idx3)
    x = gathered.reshape(BATCH, EMBED_DIM)
    labels3 = labels.astype(jnp.int32).reshape(NB, 1, BM)
    loss = _tc_loss(x, W, b.reshape(1, NUM_CLASSES), labels3)
    return loss[0, 0]
